# baseline (speedup 1.0000x reference)
import functools

import jax
import jax.numpy as jnp
from jax import lax
from jax.experimental import pallas as pl
from jax.experimental.pallas import tpu as pltpu
from jax.experimental.pallas import tpu_sc as plsc

D = 128
DW = D // 2
LANES = 16
NW = 32
C = 80


def _pack_bf16(a):
    n = a.shape[0]
    return lax.bitcast_convert_type(
        a.astype(jnp.bfloat16).reshape(n, DW, 2), jnp.float32)


def _sc_decode(zp, src_idx, dst_idx, rel_idx, wp, n_edges):
    epw = n_edges // NW
    n_chunks = epw // C

    mesh = plsc.VectorSubcoreMesh(core_axis_name="c", subcore_axis_name="s")

    @functools.partial(
        pl.kernel,
        out_type=jax.ShapeDtypeStruct((n_edges,), jnp.float32),
        mesh=mesh,
        compiler_params=pltpu.CompilerParams(needs_layout_passes=False,
                                             use_tc_tiling_on_sc=False),
        scratch_types=[
            pltpu.VMEM((epw,), jnp.int32),
            pltpu.VMEM((epw,), jnp.int32),
            pltpu.VMEM((epw,), jnp.int32),
            pltpu.VMEM((C, DW), jnp.float32),
            pltpu.VMEM((C, DW), jnp.float32),
            pltpu.VMEM((C, DW), jnp.float32),
            pltpu.VMEM((C, DW), jnp.float32),
            pltpu.VMEM((C, DW), jnp.float32),
            pltpu.VMEM((C, DW), jnp.float32),
            pltpu.VMEM((LANES, LANES), jnp.float32),
            pltpu.VMEM((LANES, LANES), jnp.float32),
            pltpu.VMEM((C,), jnp.float32),
            pltpu.VMEM((C,), jnp.float32),
            pltpu.SemaphoreType.DMA,
            pltpu.SemaphoreType.DMA,
            pltpu.SemaphoreType.DMA,
        ],
    )
    def decode(z_hbm, src_hbm, dst_hbm, rel_hbm, w_hbm, out_hbm,
               si_v, di_v, ri_v, sr0, dr0, rr0, sr1, dr1, rr1,
               t_va, t_vb, ob0, ob1, sem0, sem1, sem_o):
        wid = lax.axis_index("s") * 2 + lax.axis_index("c")
        base0 = wid * epw
        iota = lax.iota(jnp.int32, LANES)

        pltpu.sync_copy(src_hbm.at[pl.ds(base0, epw)], si_v)
        pltpu.sync_copy(dst_hbm.at[pl.ds(base0, epw)], di_v)
        pltpu.sync_copy(rel_hbm.at[pl.ds(base0, epw)], ri_v)

        def row_copies(g, sr, dr, rr, sem):
            off = g * C
            return (
                pltpu.make_async_copy(z_hbm.at[si_v.at[pl.ds(off, C)]], sr, sem),
                pltpu.make_async_copy(z_hbm.at[di_v.at[pl.ds(off, C)]], dr, sem),
                pltpu.make_async_copy(w_hbm.at[ri_v.at[pl.ds(off, C)]], rr, sem),
            )

        def issue(g, sr, dr, rr, sem):
            for cp in row_copies(g, sr, dr, rr, sem):
                cp.start()

        def wait(g, sr, dr, rr, sem):
            for cp in row_copies(g, sr, dr, rr, sem):
                cp.wait()

        def edge_loads(sr, dr, rr, gb, k):
            return [(sr[gb + k, pl.ds(j * LANES, LANES)],
                     dr[gb + k, pl.ds(j * LANES, LANES)],
                     rr[gb + k, pl.ds(j * LANES, LANES)])
                    for j in range(DW // LANES)]

        def edge_score(loaded):
            halves = []
            for sv, dv, wv in loaded:
                p = (plsc.bitcast(sv, jnp.bfloat16)
                     * plsc.bitcast(dv, jnp.bfloat16)
                     * plsc.bitcast(wv, jnp.bfloat16))
                p0, p1 = plsc.unpack(p, format=plsc.PackFormat.INTERLEAVED)
                halves += [p0, p1]
            while len(halves) > 1:
                halves = [a + b for a, b in zip(halves[0::2], halves[1::2])]
            return halves[0]

        def out_copy(g, ob):
            return pltpu.make_async_copy(
                ob, out_hbm.at[pl.ds(base0 + g * C, C)], sem_o)

        def group_score(sr, dr, rr, gb, t_v, ob):
            cur = edge_loads(sr, dr, rr, gb, 0)
            for k in range(LANES):
                nxt = (edge_loads(sr, dr, rr, gb, k + 1)
                       if k + 1 < LANES else None)
                t_v[k, :] = edge_score(cur)
                cur = nxt

            cols = [plsc.load_gather(
                        t_v, [iota, jnp.full((LANES,), k, jnp.int32)])
                    for k in range(LANES)]
            while len(cols) > 1:
                cols = [a + b for a, b in zip(cols[0::2], cols[1::2])]
            s = 1.0 / (1.0 + jnp.exp(-cols[0]))
            ob[pl.ds(gb, LANES)] = s

        def score_chunk(g, sr, dr, rr, ob):
            for grp in range(C // LANES):
                group_score(sr, dr, rr, grp * LANES,
                            t_va if grp % 2 == 0 else t_vb, ob)
            out_copy(g, ob).start()

        bufs = ((sr0, dr0, rr0, sem0), (sr1, dr1, rr1, sem1))
        obs = (ob0, ob1)
        issue(0, *bufs[0])

        def pair_body(i, carry):
            g = 2 * i
            for p in range(2):
                gc = g + p
                issue(gc + 1, *bufs[1 - p])
                wait(gc, *bufs[p])

                @pl.when(gc >= 2)
                def _():
                    out_copy(gc - 2, obs[p]).wait()

                score_chunk(gc, *bufs[p][:3], obs[p])
            return carry

        lax.fori_loop(0, (n_chunks - 1) // 2, pair_body, 0)
        last = n_chunks - 1
        wait(last, *bufs[last % 2])
        out_copy(last - 2, obs[last % 2]).wait()
        score_chunk(last, *bufs[last % 2][:3], obs[last % 2])
        out_copy(n_chunks - 2, obs[(n_chunks - 2) % 2]).wait()
        out_copy(last, obs[last % 2]).wait()

    return decode(zp, src_idx, dst_idx, rel_idx, wp)


def kernel(z, edge_index, edge_type, weight):
    n_edges = edge_index.shape[1]
    src_idx = edge_index[0]
    dst_idx = edge_index[1]
    return _sc_decode(_pack_bf16(z), src_idx, dst_idx, edge_type,
                      _pack_bf16(weight), n_edges)


# speedup vs baseline: 1.2514x; 1.2514x over previous
import functools

import jax
import jax.numpy as jnp
from jax import lax
from jax.experimental import pallas as pl
from jax.experimental.pallas import tpu as pltpu
from jax.experimental.pallas import tpu_sc as plsc

D = 128
DW = D // 2
LANES = 16
NW = 32
C = 80


def _pack_bf16(a):
    n = a.shape[0]
    return lax.bitcast_convert_type(
        a.astype(jnp.bfloat16).reshape(n, DW, 2), jnp.float32)


def _sc_decode(zp, src_idx, dst_idx, rel_idx, wp, n_edges):
    epw = n_edges // NW
    n_chunks = epw // C

    mesh = plsc.VectorSubcoreMesh(core_axis_name="c", subcore_axis_name="s")

    @functools.partial(
        pl.kernel,
        out_type=jax.ShapeDtypeStruct((n_edges,), jnp.float32),
        mesh=mesh,
        compiler_params=pltpu.CompilerParams(needs_layout_passes=False,
                                             use_tc_tiling_on_sc=False),
        scratch_types=[
            pltpu.VMEM((epw,), jnp.int32),
            pltpu.VMEM((epw,), jnp.int32),
            pltpu.VMEM((epw,), jnp.int32),
            pltpu.VMEM((C, DW), jnp.float32),
            pltpu.VMEM((C, DW), jnp.float32),
            pltpu.VMEM((C, DW), jnp.float32),
            pltpu.VMEM((C, DW), jnp.float32),
            pltpu.VMEM((C, DW), jnp.float32),
            pltpu.VMEM((C, DW), jnp.float32),
            pltpu.VMEM((C, DW), jnp.float32),
            pltpu.VMEM((C, DW), jnp.float32),
            pltpu.VMEM((C, DW), jnp.float32),
            pltpu.VMEM((LANES, LANES), jnp.float32),
            pltpu.VMEM((C,), jnp.float32),
            pltpu.VMEM((C,), jnp.float32),
            pltpu.VMEM((C,), jnp.float32),
            pltpu.SemaphoreType.DMA,
            pltpu.SemaphoreType.DMA,
            pltpu.SemaphoreType.DMA,
            pltpu.SemaphoreType.DMA,
        ],
    )
    def decode(z_hbm, src_hbm, dst_hbm, rel_hbm, w_hbm, out_hbm,
               si_v, di_v, ri_v, sr0, dr0, rr0, sr1, dr1, rr1,
               sr2, dr2, rr2, t_v, ob0, ob1, ob2, sem0, sem1, sem2, sem_o):
        wid = lax.axis_index("s") * 2 + lax.axis_index("c")
        base0 = wid * epw
        iota = lax.iota(jnp.int32, LANES)

        pltpu.sync_copy(src_hbm.at[pl.ds(base0, epw)], si_v)
        pltpu.sync_copy(dst_hbm.at[pl.ds(base0, epw)], di_v)
        pltpu.sync_copy(rel_hbm.at[pl.ds(base0, epw)], ri_v)

        def row_copies(g, sr, dr, rr, sem):
            off = g * C
            return (
                pltpu.make_async_copy(z_hbm.at[si_v.at[pl.ds(off, C)]], sr, sem),
                pltpu.make_async_copy(z_hbm.at[di_v.at[pl.ds(off, C)]], dr, sem),
                pltpu.make_async_copy(w_hbm.at[ri_v.at[pl.ds(off, C)]], rr, sem),
            )

        def issue(g, sr, dr, rr, sem):
            for cp in row_copies(g, sr, dr, rr, sem):
                cp.start()

        def wait(g, sr, dr, rr, sem):
            for cp in row_copies(g, sr, dr, rr, sem):
                cp.wait()

        def edge_loads(sr, dr, rr, gb, k):
            return [(sr[gb + k, pl.ds(j * LANES, LANES)],
                     dr[gb + k, pl.ds(j * LANES, LANES)],
                     rr[gb + k, pl.ds(j * LANES, LANES)])
                    for j in range(DW // LANES)]

        def edge_score(loaded):
            halves = []
            for sv, dv, wv in loaded:
                p = (plsc.bitcast(sv, jnp.bfloat16)
                     * plsc.bitcast(dv, jnp.bfloat16)
                     * plsc.bitcast(wv, jnp.bfloat16))
                p0, p1 = plsc.unpack(p, format=plsc.PackFormat.INTERLEAVED)
                halves += [p0, p1]
            while len(halves) > 1:
                halves = [a + b for a, b in zip(halves[0::2], halves[1::2])]
            return halves[0]

        def out_copy(g, ob):
            return pltpu.make_async_copy(
                ob, out_hbm.at[pl.ds(base0 + g * C, C)], sem_o)

        def lane_sum(x):
            for m in (1, 2, 4, 8):
                x = x + x.at[iota ^ m].get(mode="promise_in_bounds")
            return x

        def score_chunk(g, sr, dr, rr, ob):
            def group_body(grp, carry):
                gb = grp * LANES

                cur = edge_loads(sr, dr, rr, gb, 0)
                s = None
                for k in range(LANES):
                    nxt = (edge_loads(sr, dr, rr, gb, k + 1)
                           if k + 1 < LANES else None)
                    tot = lane_sum(edge_score(cur))
                    s = tot if s is None else jnp.where(iota == k, tot, s)
                    cur = nxt

                s = 1.0 / (1.0 + jnp.exp(-s))
                ob[pl.ds(gb, LANES)] = s
                return carry

            lax.fori_loop(0, C // LANES, group_body, 0)
            out_copy(g, ob).start()

        bufs = ((sr0, dr0, rr0, sem0), (sr1, dr1, rr1, sem1),
                (sr2, dr2, rr2, sem2))
        obs = (ob0, ob1, ob2)
        issue(0, *bufs[0])
        issue(1, *bufs[1])

        def triple_body(i, carry):
            g = 3 * i
            for p in range(3):
                gc = g + p
                issue(gc + 2, *bufs[(p + 2) % 3])
                wait(gc, *bufs[p])

                @pl.when(gc >= 3)
                def _():
                    out_copy(gc - 3, obs[p]).wait()

                score_chunk(gc, *bufs[p][:3], obs[p])
            return carry

        lax.fori_loop(0, (n_chunks - 2) // 3, triple_body, 0)
        for g in range(n_chunks - 2, n_chunks):
            wait(g, *bufs[g % 3])
            out_copy(g - 3, obs[g % 3]).wait()
            score_chunk(g, *bufs[g % 3][:3], obs[g % 3])
        for g in range(n_chunks - 3, n_chunks):
            out_copy(g, obs[g % 3]).wait()

    return decode(zp, src_idx, dst_idx, rel_idx, wp)


def kernel(z, edge_index, edge_type, weight):
    n_edges = edge_index.shape[1]
    src_idx = edge_index[0]
    dst_idx = edge_index[1]
    return _sc_decode(_pack_bf16(z), src_idx, dst_idx, edge_type,
                      _pack_bf16(weight), n_edges)


# speedup vs baseline: 1.6452x; 1.3147x over previous
import functools

import jax
import jax.numpy as jnp
from jax import lax
from jax.experimental import pallas as pl
from jax.experimental.pallas import tpu as pltpu
from jax.experimental.pallas import tpu_sc as plsc

D = 128
DW = D // 2
LANES = 16
NW = 32
C = 80


def _pack_bf16(a):
    n = a.shape[0]
    return lax.bitcast_convert_type(
        a.astype(jnp.bfloat16).reshape(n, DW, 2), jnp.float32)


def _sc_decode(zp, src_idx, dst_idx, rel_idx, wp, n_edges):
    epw = n_edges // NW
    n_chunks = epw // C

    mesh = plsc.VectorSubcoreMesh(core_axis_name="c", subcore_axis_name="s")

    @functools.partial(
        pl.kernel,
        out_type=jax.ShapeDtypeStruct((n_edges,), jnp.float32),
        mesh=mesh,
        compiler_params=pltpu.CompilerParams(needs_layout_passes=False,
                                             use_tc_tiling_on_sc=False),
        scratch_types=[
            pltpu.VMEM((epw,), jnp.int32),
            pltpu.VMEM((epw,), jnp.int32),
            pltpu.VMEM((epw,), jnp.int32),
            pltpu.VMEM((C, DW), jnp.float32),
            pltpu.VMEM((C, DW), jnp.float32),
            pltpu.VMEM((C, DW), jnp.float32),
            pltpu.VMEM((C, DW), jnp.float32),
            pltpu.VMEM((C, DW), jnp.float32),
            pltpu.VMEM((C, DW), jnp.float32),
            pltpu.VMEM((C, DW), jnp.float32),
            pltpu.VMEM((C, DW), jnp.float32),
            pltpu.VMEM((C, DW), jnp.float32),
            pltpu.VMEM((LANES, LANES), jnp.float32),
            pltpu.VMEM((C,), jnp.float32),
            pltpu.VMEM((C,), jnp.float32),
            pltpu.VMEM((C,), jnp.float32),
            pltpu.SemaphoreType.DMA,
            pltpu.SemaphoreType.DMA,
            pltpu.SemaphoreType.DMA,
            pltpu.SemaphoreType.DMA,
        ],
    )
    def decode(z_hbm, src_hbm, dst_hbm, rel_hbm, w_hbm, out_hbm,
               si_v, di_v, ri_v, sr0, dr0, rr0, sr1, dr1, rr1,
               sr2, dr2, rr2, t_v, ob0, ob1, ob2, sem0, sem1, sem2, sem_o):
        wid = lax.axis_index("s") * 2 + lax.axis_index("c")
        base0 = wid * epw
        iota = lax.iota(jnp.int32, LANES)

        pltpu.sync_copy(src_hbm.at[pl.ds(base0, epw)], si_v)
        pltpu.sync_copy(dst_hbm.at[pl.ds(base0, epw)], di_v)
        pltpu.sync_copy(rel_hbm.at[pl.ds(base0, epw)], ri_v)

        def row_copies(g, sr, dr, rr, sem):
            off = g * C
            return (
                pltpu.make_async_copy(z_hbm.at[si_v.at[pl.ds(off, C)]], sr, sem),
                pltpu.make_async_copy(z_hbm.at[di_v.at[pl.ds(off, C)]], dr, sem),
                pltpu.make_async_copy(w_hbm.at[ri_v.at[pl.ds(off, C)]], rr, sem),
            )

        def issue(g, sr, dr, rr, sem):
            for cp in row_copies(g, sr, dr, rr, sem):
                cp.start()

        def wait(g, sr, dr, rr, sem):
            for cp in row_copies(g, sr, dr, rr, sem):
                cp.wait()

        def edge_loads(sr, dr, rr, gb, k):
            return [(sr[gb + k, pl.ds(j * LANES, LANES)],
                     dr[gb + k, pl.ds(j * LANES, LANES)],
                     rr[gb + k, pl.ds(j * LANES, LANES)])
                    for j in range(DW // LANES)]

        def edge_score(loaded):
            prods = []
            for sv, dv, wv in loaded:
                prods.append(plsc.bitcast(sv, jnp.bfloat16)
                             * plsc.bitcast(dv, jnp.bfloat16)
                             * plsc.bitcast(wv, jnp.bfloat16))
            while len(prods) > 1:
                prods = [a + b for a, b in zip(prods[0::2], prods[1::2])]
            p0, p1 = plsc.unpack(prods[0], format=plsc.PackFormat.INTERLEAVED)
            return p0 + p1

        def out_copy(g, ob):
            return pltpu.make_async_copy(
                ob, out_hbm.at[pl.ds(base0 + g * C, C)], sem_o)

        def score_chunk(g, sr, dr, rr, ob):
            def group_body(grp, carry):
                gb = grp * LANES

                cur = edge_loads(sr, dr, rr, gb, 0)
                for k in range(LANES):
                    nxt = (edge_loads(sr, dr, rr, gb, k + 1)
                           if k + 1 < LANES else None)
                    t_v[k, :] = edge_score(cur)
                    cur = nxt

                cols = [plsc.load_gather(
                            t_v, [iota, jnp.full((LANES,), k, jnp.int32)])
                        for k in range(LANES)]
                while len(cols) > 1:
                    cols = [a + b for a, b in zip(cols[0::2], cols[1::2])]
                s = 1.0 / (1.0 + jnp.exp(-cols[0]))
                ob[pl.ds(gb, LANES)] = s
                return carry

            lax.fori_loop(0, C // LANES, group_body, 0)
            out_copy(g, ob).start()

        bufs = ((sr0, dr0, rr0, sem0), (sr1, dr1, rr1, sem1),
                (sr2, dr2, rr2, sem2))
        obs = (ob0, ob1, ob2)
        issue(0, *bufs[0])
        issue(1, *bufs[1])

        def triple_body(i, carry):
            g = 3 * i
            for p in range(3):
                gc = g + p
                issue(gc + 2, *bufs[(p + 2) % 3])
                wait(gc, *bufs[p])

                @pl.when(gc >= 3)
                def _():
                    out_copy(gc - 3, obs[p]).wait()

                score_chunk(gc, *bufs[p][:3], obs[p])
            return carry

        lax.fori_loop(0, (n_chunks - 2) // 3, triple_body, 0)
        for g in range(n_chunks - 2, n_chunks):
            wait(g, *bufs[g % 3])
            out_copy(g - 3, obs[g % 3]).wait()
            score_chunk(g, *bufs[g % 3][:3], obs[g % 3])
        for g in range(n_chunks - 3, n_chunks):
            out_copy(g, obs[g % 3]).wait()

    return decode(zp, src_idx, dst_idx, rel_idx, wp)


def kernel(z, edge_index, edge_type, weight):
    n_edges = edge_index.shape[1]
    src_idx = edge_index[0]
    dst_idx = edge_index[1]
    return _sc_decode(_pack_bf16(z), src_idx, dst_idx, edge_type,
                      _pack_bf16(weight), n_edges)
